# baseline (speedup 1.0000x reference)
import functools

import jax
import jax.numpy as jnp
from jax import lax
from jax.experimental import pallas as pl
from jax.experimental.pallas import tpu as pltpu
from jax.experimental.pallas import tpu_sc as plsc

N = 10000
D = 128
H = 8
DH = 16
E = 320000
EL = 200000
LANES = 16
NC = 2
NS = 16
NW = NC * NS

GRP = 128
IGRP = 8
S1_CHUNK = GRP * IGRP
S1_QTR = S1_CHUNK // 4
E_PAD = 327680
S1_CPW = E_PAD // S1_CHUNK // NW
NP = N + 8

NSR = 1280
S1_BATCH = 32
S1_SETS = 2
S1_BPC = S1_CHUNK // S1_BATCH
ZROWS = 8
ZBLOCKS = N // ZROWS
ZBLOCKS_S = NSR // ZROWS
DROWS = 80
DBLOCKS = N // DROWS
DBLOCKS_S = NSR // DROWS

S2_CHUNK = GRP * IGRP
S2_QTR = S2_CHUNK // 4
EL_PAD = 200704
S2_NCHUNKS = EL_PAD // S2_CHUNK
S2_ITERS = -(-S2_NCHUNKS // NW)

TBLK = 1000


def _leaky(x):
    return jnp.where(x > 0, x, x * 0.2)


def _t1_body(x_ref, w_ref, b_ref, am1_ref, am2_ref,
             h_ref, a1_ref, a2_ref, m1_ref, m2_ref):
    i = pl.program_id(0)
    h = jnp.dot(x_ref[...], w_ref[...], preferred_element_type=jnp.float32)
    h = h + b_ref[...]
    h_ref[...] = h
    a1 = jnp.dot(h, am1_ref[...], preferred_element_type=jnp.float32)
    a2 = jnp.dot(h, am2_ref[...], preferred_element_type=jnp.float32)
    a1_ref[...] = a1
    a2_ref[...] = a2
    bm1 = jnp.max(a1, axis=0, keepdims=True)
    bm2 = jnp.max(a2, axis=0, keepdims=True)

    @pl.when(i == 0)
    def _():
        m1_ref[...] = bm1
        m2_ref[...] = bm2

    @pl.when(i != 0)
    def _():
        m1_ref[...] = jnp.maximum(m1_ref[...], bm1)
        m2_ref[...] = jnp.maximum(m2_ref[...], bm2)


def _project(x, w, b, am1, am2):
    return pl.pallas_call(
        _t1_body,
        grid=(N // TBLK,),
        in_specs=[
            pl.BlockSpec((TBLK, D), lambda i: (i, 0)),
            pl.BlockSpec((D, D), lambda i: (0, 0)),
            pl.BlockSpec((1, D), lambda i: (0, 0)),
            pl.BlockSpec((D, D), lambda i: (0, 0)),
            pl.BlockSpec((D, D), lambda i: (0, 0)),
        ],
        out_specs=[
            pl.BlockSpec((TBLK, D), lambda i: (i, 0)),
            pl.BlockSpec((TBLK, D), lambda i: (i, 0)),
            pl.BlockSpec((TBLK, D), lambda i: (i, 0)),
            pl.BlockSpec((1, D), lambda i: (0, 0)),
            pl.BlockSpec((1, D), lambda i: (0, 0)),
        ],
        out_shape=[
            jax.ShapeDtypeStruct((N, D), jnp.float32),
            jax.ShapeDtypeStruct((N, D), jnp.float32),
            jax.ShapeDtypeStruct((N, D), jnp.float32),
            jax.ShapeDtypeStruct((1, D), jnp.float32),
            jax.ShapeDtypeStruct((1, D), jnp.float32),
        ],
    )(x, w, b, am1, am2)


@functools.cache
def _sc_mesh():
    return plsc.VectorSubcoreMesh(
        core_axis_name="core", subcore_axis_name="subcore",
        num_cores=NC, num_subcores=NS)


@functools.cache
def _s1_kernel():
    return pl.kernel(
        _s1_conv,
        out_type=(
            jax.ShapeDtypeStruct((NC, N, D), jnp.float32),
            jax.ShapeDtypeStruct((NC, NSR, D), jnp.float32),
            jax.ShapeDtypeStruct((NC, N, D), jnp.float32),
            jax.ShapeDtypeStruct((NC, NSR, D), jnp.float32),
        ),
        mesh=_sc_mesh(),
        scratch_types=[
            pltpu.VMEM_SHARED((NP, D), jnp.float32),
            pltpu.VMEM_SHARED((NSR, D), jnp.float32),
            pltpu.VMEM((IGRP, GRP), jnp.int32),
            pltpu.VMEM((IGRP, GRP), jnp.int32),
            pltpu.VMEM((S1_SETS, S1_BATCH, D), jnp.float32),
            pltpu.VMEM((S1_SETS, S1_BATCH, D), jnp.float32),
            pltpu.VMEM((S1_SETS, S1_BATCH, LANES), jnp.float32),
            pltpu.VMEM((S1_SETS, S1_BATCH, D), jnp.float32),
            pltpu.VMEM((D,), jnp.float32),
            pltpu.VMEM((ZROWS, D), jnp.float32),
            pltpu.SemaphoreType.DMA((S1_SETS,)),
            pltpu.SemaphoreType.DMA((S1_SETS,)),
        ],
        compiler_params=pltpu.CompilerParams(needs_layout_passes=False),
    )


def _s1_conv(hreq_hbm, hcode_hbm, asf_hbm, adf_hbm, asr_hbm, adr_hbm,
             srcf_hbm, dstf_hbm, srcr_hbm, dstr_hbm, mf_hbm, mr_hbm,
             pf_hbm, sf_hbm, pr_hbm, sr_hbm,
             out_acc, s_acc, srcv, dstv, asb, adb, wb,
             msgb, m_v, zbuf, sem_ga, sem_sa):
    core = lax.axis_index("core")
    sub = lax.axis_index("subcore")
    wid = sub * NC + core

    zero = jnp.zeros((LANES,), jnp.float32)
    lanes_iota = lax.iota(jnp.int32, LANES)

    @pl.loop(0, ZROWS)
    def _(r):
        @pl.loop(0, D, step=LANES)
        def _(c):
            zbuf[r, pl.ds(c, LANES)] = zero

    for direction in range(2):
        if direction == 0:
            h_src, a_src, a_dst = hreq_hbm, asf_hbm, adf_hbm
            src_e, dst_e, m_h = srcf_hbm, dstf_hbm, mf_hbm
            p_out, s_out = pf_hbm, sf_hbm
        else:
            h_src, a_src, a_dst = hcode_hbm, asr_hbm, adr_hbm
            src_e, dst_e, m_h = srcr_hbm, dstr_hbm, mr_hbm
            p_out, s_out = pr_hbm, sr_hbm

        @pl.loop(0, -(-(ZBLOCKS + ZBLOCKS_S) // NS))
        def _(t):
            b = t * NS + sub

            @pl.when(b < ZBLOCKS)
            def _():
                pltpu.sync_copy(zbuf, out_acc.at[pl.ds(b * ZROWS, ZROWS)])

            @pl.when(jnp.logical_and(b >= ZBLOCKS, b < ZBLOCKS + ZBLOCKS_S))
            def _():
                pltpu.sync_copy(
                    zbuf, s_acc.at[pl.ds((b - ZBLOCKS) * ZROWS, ZROWS)])

        pltpu.sync_copy(m_h, m_v)
        mreg = m_v[pl.ds(0, LANES)]
        plsc.subcore_barrier()


        def issue_gathers(b, s):
            row = lax.shift_right_logical(b, 2)
            off = lax.bitwise_and(b, 3) * S1_BATCH
            for k in range(S1_BATCH // LANES):
                sreg = srcv[row, pl.ds(off + k * LANES, LANES)]
                dreg = dstv[row, pl.ds(off + k * LANES, LANES)]
                sl = pl.ds(k * LANES, LANES)
                pltpu.async_copy(a_src.at[sreg], asb.at[s, sl], sem_ga.at[s])
                pltpu.async_copy(a_dst.at[dreg], adb.at[s, sl], sem_ga.at[s])
                pltpu.async_copy(h_src.at[sreg], msgb.at[s, sl], sem_ga.at[s])

        def wait_gathers(s):
            pltpu.make_async_copy(
                a_src.at[pl.ds(0, S1_BATCH)], asb.at[s], sem_ga.at[s]).wait()
            pltpu.make_async_copy(
                a_dst.at[pl.ds(0, S1_BATCH)], adb.at[s], sem_ga.at[s]).wait()
            pltpu.make_async_copy(
                h_src.at[pl.ds(0, S1_BATCH)], msgb.at[s], sem_ga.at[s]).wait()

        def wait_scatters(s):
            pltpu.make_async_copy(
                msgb.at[s], out_acc.at[pl.ds(0, S1_BATCH)], sem_sa.at[s]).wait()
            pltpu.make_async_copy(
                adb.at[s], s_acc.at[pl.ds(0, S1_BATCH)], sem_sa.at[s]).wait()

        @pl.loop(0, S1_CPW)
        def _(t):
            cid = wid * S1_CPW + t
            g0 = cid * IGRP
            pltpu.sync_copy(src_e.at[pl.ds(g0, IGRP)], srcv)
            pltpu.sync_copy(dst_e.at[pl.ds(g0, IGRP)], dstv)

            issue_gathers(0, 0)

            @pl.loop(0, S1_BPC)
            def _(b):
                s = lax.rem(b, S1_SETS)
                sn = lax.rem(b + 1, S1_SETS)

                @pl.when(b + 1 < S1_BPC)
                def _():
                    @pl.when(b >= S1_SETS - 1)
                    def _():
                        wait_scatters(sn)
                    issue_gathers(b + 1, sn)

                row = lax.shift_right_logical(b, 2)
                off = lax.bitwise_and(b, 3) * S1_BATCH
                dreg = dstv[row, pl.ds(off, LANES)]
                dreg2 = dstv[row, pl.ds(off + LANES, LANES)]
                wait_gathers(s)

                @pl.loop(0, S1_BATCH)
                def _(i):
                    x = (asb[s, i, pl.ds(0, LANES)]
                         + adb[s, i, pl.ds(0, LANES)])
                    wb[s, i, :] = jnp.exp(_leaky(x) - mreg)

                sv = jnp.full((LANES,), s, jnp.int32)

                @pl.loop(0, S1_BATCH)
                def _(i):
                    dsts = plsc.load_gather(
                        dstv, [jnp.full((LANES,), row, jnp.int32),
                               jnp.full((LANES,), off + i, jnp.int32)])
                    col = (lax.bitwise_and(dsts, 7) * LANES) + lanes_iota
                    for blk in range(H):
                        adb[s, i, pl.ds(blk * LANES, LANES)] = zero
                    plsc.store_scatter(
                        adb, [sv, jnp.full((LANES,), i, jnp.int32), col],
                        wb[s, i, :])
                    iv = jnp.full((LANES,), i, jnp.int32)
                    for j in range(H):
                        jv = jnp.full((LANES,), j, jnp.int32)
                        wspl = plsc.load_gather(wb, [sv, iv, jv])
                        sl = pl.ds(j * DH, DH)
                        msgb[s, i, sl] = msgb[s, i, sl] * wspl

                for k, dr in enumerate((dreg, dreg2)):
                    sl = pl.ds(k * LANES, LANES)
                    d8 = lax.shift_right_logical(dr, 3)
                    pltpu.async_copy(msgb.at[s, sl], out_acc.at[dr],
                                     sem_sa.at[s], add=True)
                    pltpu.async_copy(adb.at[s, sl], s_acc.at[d8],
                                     sem_sa.at[s], add=True)

            for sst in range(S1_SETS):
                wait_scatters(jnp.int32(sst))

        plsc.subcore_barrier()

        @pl.loop(0, -(-(DBLOCKS + DBLOCKS_S) // NS))
        def _(t):
            b = t * NS + sub

            @pl.when(b < DBLOCKS)
            def _():
                r = b * DROWS
                pltpu.sync_copy(out_acc.at[pl.ds(r, DROWS)],
                                p_out.at[core, pl.ds(r, DROWS)])

            @pl.when(jnp.logical_and(b >= DBLOCKS, b < DBLOCKS + DBLOCKS_S))
            def _():
                r = (b - DBLOCKS) * DROWS
                pltpu.sync_copy(s_acc.at[pl.ds(r, DROWS)],
                                s_out.at[core, pl.ds(r, DROWS)])

        plsc.subcore_barrier()


def _t2a_body(p0_ref, p1_ref, s0_ref, s1_ref, exp_ref,
              o_ref, sum1_ref, sum2_ref):
    i = pl.program_id(0)
    t = p0_ref[0] + p1_ref[0]
    ssum = s0_ref[0] + s1_ref[0]
    den = jnp.dot(ssum, exp_ref[...], preferred_element_type=jnp.float32)
    o = jnp.maximum(t / (den + 1e-16), 0.0)
    o_ref[...] = o
    b1 = jnp.sum(o, axis=0, keepdims=True)
    b2 = jnp.sum(o * o, axis=0, keepdims=True)

    @pl.when(i == 0)
    def _():
        sum1_ref[...] = b1
        sum2_ref[...] = b2

    @pl.when(i != 0)
    def _():
        sum1_ref[...] = sum1_ref[...] + b1
        sum2_ref[...] = sum2_ref[...] + b2


def _t2a_call(p, s, expand):
    return pl.pallas_call(
        _t2a_body,
        grid=(N // TBLK,),
        in_specs=[
            pl.BlockSpec((1, TBLK, D), lambda i: (0, i, 0)),
            pl.BlockSpec((1, TBLK, D), lambda i: (1, i, 0)),
            pl.BlockSpec((1, TBLK, LANES), lambda i: (0, i, 0)),
            pl.BlockSpec((1, TBLK, LANES), lambda i: (1, i, 0)),
            pl.BlockSpec((LANES, D), lambda i: (0, 0)),
        ],
        out_specs=[
            pl.BlockSpec((TBLK, D), lambda i: (i, 0)),
            pl.BlockSpec((1, D), lambda i: (0, 0)),
            pl.BlockSpec((1, D), lambda i: (0, 0)),
        ],
        out_shape=[
            jax.ShapeDtypeStruct((N, D), jnp.float32),
            jax.ShapeDtypeStruct((1, D), jnp.float32),
            jax.ShapeDtypeStruct((1, D), jnp.float32),
        ],
    )(p, p, s, s, expand)


def _t2b_body(o_ref, a_ref, b_ref, out_ref):
    out_ref[...] = o_ref[...] * a_ref[...] + b_ref[...]


def _bn_apply(o, a, b):
    return pl.pallas_call(
        _t2b_body,
        grid=(N // TBLK,),
        in_specs=[
            pl.BlockSpec((TBLK, D), lambda i: (i, 0)),
            pl.BlockSpec((1, D), lambda i: (0, 0)),
            pl.BlockSpec((1, D), lambda i: (0, 0)),
        ],
        out_specs=pl.BlockSpec((TBLK, D), lambda i: (i, 0)),
        out_shape=jax.ShapeDtypeStruct((N, D), jnp.float32),
    )(o, a, b)


@functools.cache
def _s2_kernel():
    return pl.kernel(
        _s2_score,
        out_type=jax.ShapeDtypeStruct((EL_PAD,), jnp.float32),
        mesh=_sc_mesh(),
        scratch_types=[
            pltpu.VMEM((IGRP, GRP), jnp.int32),
            pltpu.VMEM((IGRP, GRP), jnp.int32),
            pltpu.VMEM((S2_QTR, D), jnp.float32),
            pltpu.VMEM((S2_QTR, D), jnp.float32),
            pltpu.VMEM((S2_QTR, LANES), jnp.float32),
            pltpu.VMEM((S2_QTR,), jnp.float32),
            pltpu.SemaphoreType.DMA,
            pltpu.SemaphoreType.DMA,
        ],
        compiler_params=pltpu.CompilerParams(needs_layout_passes=False),
    )


def _s2_score(nr_hbm, ncd_hbm, i0_hbm, i1_hbm, out_hbm,
              i0, i1, r_rows, c_rows, tbuf, outbuf, sem1, sem2):
    core = lax.axis_index("core")
    sub = lax.axis_index("subcore")
    wid = sub * NC + core
    lanes_iota = lax.iota(jnp.int32, LANES)

    @pl.loop(0, S2_ITERS)
    def _(t):
        cid = t * NW + wid

        @pl.when(cid < S2_NCHUNKS)
        def _():
            g0 = cid * IGRP
            pltpu.sync_copy(i0_hbm.at[pl.ds(g0, IGRP)], i0)
            pltpu.sync_copy(i1_hbm.at[pl.ds(g0, IGRP)], i1)
            for q in range(4):
                cps = []
                for g in range(2):
                    gi = q * 2 + g
                    sl = pl.ds(g * GRP, GRP)
                    cps.append(pltpu.async_copy(nr_hbm.at[i0.at[gi]],
                                                r_rows.at[sl], sem1))
                    cps.append(pltpu.async_copy(ncd_hbm.at[i1.at[gi]],
                                                c_rows.at[sl], sem2))
                for cp in cps:
                    cp.wait()

                @pl.loop(0, S2_QTR)
                def _(i):
                    acc = (r_rows[i, pl.ds(0, LANES)]
                           * c_rows[i, pl.ds(0, LANES)])
                    for j in range(1, H):
                        sl = pl.ds(j * DH, LANES)
                        acc = acc + r_rows[i, sl] * c_rows[i, sl]
                    tbuf[i, :] = acc

                @pl.loop(0, S2_QTR, step=LANES)
                def _(g):
                    rows = lanes_iota + g
                    acc = plsc.load_gather(
                        tbuf, [rows, jnp.zeros((LANES,), jnp.int32)])
                    for c in range(1, LANES):
                        acc = acc + plsc.load_gather(
                            tbuf, [rows, jnp.full((LANES,), c, jnp.int32)])
                    outbuf[pl.ds(g, LANES)] = 1.0 / (1.0 + jnp.exp(-acc))

                pltpu.sync_copy(
                    outbuf,
                    out_hbm.at[pl.ds(cid * S2_CHUNK + q * S2_QTR, S2_QTR)])


def _amat(att):
    a3 = att.reshape(H, DH)
    eye = jnp.eye(H, D, dtype=jnp.float32)
    return (a3[:, :, None] * eye[:, None, :]).reshape(H * DH, D)


def kernel(x_req, x_code, edge_index, edge_index_rev, edge_label_index,
           W_req, b_req, W_code, b_code,
           att_src_fwd, att_dst_fwd, att_src_rev, att_dst_rev,
           q_sem, Wk, bk, bn_w, bn_b):
    f32 = jnp.float32

    am_req_1 = _amat(att_src_fwd)
    am_req_2 = _amat(att_dst_rev)
    am_code_1 = _amat(att_dst_fwd)
    am_code_2 = _amat(att_src_rev)

    h_req, asf, adr, max_asf, max_adr = _project(
        x_req, W_req, b_req.reshape(1, D), am_req_1, am_req_2)
    h_code, adf, asr, max_adf, max_asr = _project(
        x_code, W_code, b_code.reshape(1, D), am_code_1, am_code_2)

    m_fwd = _leaky(max_asf + max_adf).reshape(D)
    m_rev = _leaky(max_asr + max_adr).reshape(D)

    pad_n = ((0, NP - N), (0, 0))
    h_req_p = jnp.pad(h_req, pad_n)
    h_code_p = jnp.pad(h_code, pad_n)
    asf_p = jnp.pad(asf, pad_n)
    adf_p = jnp.pad(adf, pad_n)
    asr_p = jnp.pad(asr, pad_n)
    adr_p = jnp.pad(adr, pad_n)

    def edges2d(v):
        v = jnp.pad(v, (0, E_PAD - E), constant_values=N)
        return v.reshape(E_PAD // GRP, GRP)

    srcf = edges2d(edge_index[0])
    dstf = edges2d(edge_index[1])
    srcr = edges2d(edge_index_rev[0])
    dstr = edges2d(edge_index_rev[1])

    pf, sf, pr, sr = _s1_kernel()(h_req_p, h_code_p, asf_p, adf_p, asr_p,
                                  adr_p, srcf, dstf, srcr, dstr,
                                  m_fwd, m_rev)

    def unpack_s(s):
        return s.reshape(NC, NSR * H, LANES)[:, :N, :]

    sf_u = unpack_s(sf)
    sr_u = unpack_s(sr)

    expand = jnp.concatenate(
        [jnp.eye(H, dtype=f32), jnp.zeros((LANES - H, H), f32)], axis=0)
    expand = jnp.repeat(expand, DH, axis=1)

    out_code, c_sum1, c_sum2 = _t2a_call(pf, sf_u, expand)
    out_req, r_sum1, r_sum2 = _t2a_call(pr, sr_u, expand)

    def bn_coeffs(s1, s2):
        mu = s1 / N
        var = s2 / N - mu * mu
        a = bn_w.reshape(1, D) / jnp.sqrt(var + 1e-5)
        b = bn_b.reshape(1, D) - mu * a
        return a, b

    a_r, b_r = bn_coeffs(r_sum1, r_sum2)
    a_c, b_c = bn_coeffs(c_sum1, c_sum2)

    n_req = _bn_apply(out_req, a_r, b_r)
    n_code = _bn_apply(out_code, a_c, b_c)

    eli = jnp.pad(edge_label_index, ((0, 0), (0, EL_PAD - EL)))
    i0 = eli[0].reshape(EL_PAD // GRP, GRP)
    i1 = eli[1].reshape(EL_PAD // GRP, GRP)

    scores = _s2_kernel()(n_req, n_code, i0, i1)
    return scores[:EL]


# speedup vs baseline: 1.3897x; 1.3897x over previous
import functools

import jax
import jax.numpy as jnp
from jax import lax
from jax.experimental import pallas as pl
from jax.experimental.pallas import tpu as pltpu
from jax.experimental.pallas import tpu_sc as plsc

N = 10000
D = 128
H = 8
DH = 16
E = 320000
EL = 200000
LANES = 16
NC = 2
NS = 16
NW = NC * NS

GRP = 128
IGRP = 8
S1_CHUNK = GRP * IGRP
S1_QTR = S1_CHUNK // 4
E_PAD = 327680
S1_CPW = E_PAD // S1_CHUNK // NW
NP = N + 8

NSR = 1280
S1_BATCH = 16
S1_SETS = 4
S1_BPC = S1_CHUNK // S1_BATCH
ZROWS = 8
ZBLOCKS = N // ZROWS
ZBLOCKS_S = NSR // ZROWS
DROWS = 80
DBLOCKS = N // DROWS
DBLOCKS_S = NSR // DROWS

S2_CHUNK = GRP * IGRP
S2_QTR = S2_CHUNK // 4
EL_PAD = 200704
S2_NCHUNKS = EL_PAD // S2_CHUNK
S2_ITERS = -(-S2_NCHUNKS // NW)

TBLK = 1000


def _leaky(x):
    return jnp.where(x > 0, x, x * 0.2)


def _t1_body(x_ref, w_ref, b_ref, am1_ref, am2_ref,
             h_ref, a1_ref, a2_ref, m1_ref, m2_ref):
    i = pl.program_id(0)
    h = jnp.dot(x_ref[...], w_ref[...], preferred_element_type=jnp.float32)
    h = h + b_ref[...]
    h_ref[...] = h
    a1 = jnp.dot(h, am1_ref[...], preferred_element_type=jnp.float32)
    a2 = jnp.dot(h, am2_ref[...], preferred_element_type=jnp.float32)
    a1_ref[...] = a1
    a2_ref[...] = a2
    bm1 = jnp.max(a1, axis=0, keepdims=True)
    bm2 = jnp.max(a2, axis=0, keepdims=True)

    @pl.when(i == 0)
    def _():
        m1_ref[...] = bm1
        m2_ref[...] = bm2

    @pl.when(i != 0)
    def _():
        m1_ref[...] = jnp.maximum(m1_ref[...], bm1)
        m2_ref[...] = jnp.maximum(m2_ref[...], bm2)


def _project(x, w, b, am1, am2):
    return pl.pallas_call(
        _t1_body,
        grid=(N // TBLK,),
        in_specs=[
            pl.BlockSpec((TBLK, D), lambda i: (i, 0)),
            pl.BlockSpec((D, D), lambda i: (0, 0)),
            pl.BlockSpec((1, D), lambda i: (0, 0)),
            pl.BlockSpec((D, D), lambda i: (0, 0)),
            pl.BlockSpec((D, D), lambda i: (0, 0)),
        ],
        out_specs=[
            pl.BlockSpec((TBLK, D), lambda i: (i, 0)),
            pl.BlockSpec((TBLK, D), lambda i: (i, 0)),
            pl.BlockSpec((TBLK, D), lambda i: (i, 0)),
            pl.BlockSpec((1, D), lambda i: (0, 0)),
            pl.BlockSpec((1, D), lambda i: (0, 0)),
        ],
        out_shape=[
            jax.ShapeDtypeStruct((N, D), jnp.float32),
            jax.ShapeDtypeStruct((N, D), jnp.float32),
            jax.ShapeDtypeStruct((N, D), jnp.float32),
            jax.ShapeDtypeStruct((1, D), jnp.float32),
            jax.ShapeDtypeStruct((1, D), jnp.float32),
        ],
    )(x, w, b, am1, am2)


@functools.cache
def _sc_mesh():
    return plsc.VectorSubcoreMesh(
        core_axis_name="core", subcore_axis_name="subcore",
        num_cores=NC, num_subcores=NS)


@functools.cache
def _s1_kernel():
    return pl.kernel(
        _s1_conv,
        out_type=(
            jax.ShapeDtypeStruct((NC, N, D), jnp.float32),
            jax.ShapeDtypeStruct((NC, NSR, D), jnp.float32),
            jax.ShapeDtypeStruct((NC, N, D), jnp.float32),
            jax.ShapeDtypeStruct((NC, NSR, D), jnp.float32),
        ),
        mesh=_sc_mesh(),
        scratch_types=[
            pltpu.VMEM_SHARED((NP, D), jnp.float32),
            pltpu.VMEM_SHARED((NSR, D), jnp.float32),
            pltpu.VMEM((IGRP, GRP), jnp.int32),
            pltpu.VMEM((IGRP, GRP), jnp.int32),
            pltpu.VMEM((S1_SETS, S1_BATCH, D), jnp.float32),
            pltpu.VMEM((S1_SETS, S1_BATCH, D), jnp.float32),
            pltpu.VMEM((S1_SETS, S1_BATCH, D), jnp.float32),
            pltpu.VMEM((D,), jnp.float32),
            pltpu.VMEM((ZROWS, D), jnp.float32),
            pltpu.SemaphoreType.DMA((S1_SETS,)),
            pltpu.SemaphoreType.DMA((S1_SETS,)),
        ],
        compiler_params=pltpu.CompilerParams(needs_layout_passes=False),
    )


def _s1_conv(hreq_hbm, hcode_hbm, asf_hbm, adf_hbm, asr_hbm, adr_hbm,
             srcf_hbm, dstf_hbm, srcr_hbm, dstr_hbm, mf_hbm, mr_hbm,
             pf_hbm, sf_hbm, pr_hbm, sr_hbm,
             out_acc, s_acc, srcv, dstv, asb, adb,
             msgb, m_v, zbuf, sem_ga, sem_sa):
    core = lax.axis_index("core")
    sub = lax.axis_index("subcore")
    wid = sub * NC + core

    zero = jnp.zeros((LANES,), jnp.float32)
    lanes_iota = lax.iota(jnp.int32, LANES)

    @pl.loop(0, ZROWS)
    def _(r):
        @pl.loop(0, D, step=LANES)
        def _(c):
            zbuf[r, pl.ds(c, LANES)] = zero

    for direction in range(2):
        if direction == 0:
            h_src, a_src, a_dst = hreq_hbm, asf_hbm, adf_hbm
            src_e, dst_e, m_h = srcf_hbm, dstf_hbm, mf_hbm
            p_out, s_out = pf_hbm, sf_hbm
        else:
            h_src, a_src, a_dst = hcode_hbm, asr_hbm, adr_hbm
            src_e, dst_e, m_h = srcr_hbm, dstr_hbm, mr_hbm
            p_out, s_out = pr_hbm, sr_hbm

        @pl.loop(0, -(-(ZBLOCKS + ZBLOCKS_S) // NS))
        def _(t):
            b = t * NS + sub

            @pl.when(b < ZBLOCKS)
            def _():
                pltpu.sync_copy(zbuf, out_acc.at[pl.ds(b * ZROWS, ZROWS)])

            @pl.when(jnp.logical_and(b >= ZBLOCKS, b < ZBLOCKS + ZBLOCKS_S))
            def _():
                pltpu.sync_copy(
                    zbuf, s_acc.at[pl.ds((b - ZBLOCKS) * ZROWS, ZROWS)])

        pltpu.sync_copy(m_h, m_v)
        mreg = m_v[pl.ds(0, LANES)]
        plsc.subcore_barrier()


        def issue_gathers(b, s):
            row = lax.shift_right_logical(b, 3)
            off = lax.bitwise_and(b, IGRP - 1) * LANES
            sreg = srcv[row, pl.ds(off, LANES)]
            dreg = dstv[row, pl.ds(off, LANES)]
            pltpu.async_copy(a_src.at[sreg], asb.at[s], sem_ga.at[s])
            pltpu.async_copy(a_dst.at[dreg], adb.at[s], sem_ga.at[s])
            pltpu.async_copy(h_src.at[sreg], msgb.at[s], sem_ga.at[s])

        def wait_gathers(s):
            pltpu.make_async_copy(
                a_src.at[pl.ds(0, S1_BATCH)], asb.at[s], sem_ga.at[s]).wait()
            pltpu.make_async_copy(
                a_dst.at[pl.ds(0, S1_BATCH)], adb.at[s], sem_ga.at[s]).wait()
            pltpu.make_async_copy(
                h_src.at[pl.ds(0, S1_BATCH)], msgb.at[s], sem_ga.at[s]).wait()

        def wait_scatters(s):
            pltpu.make_async_copy(
                msgb.at[s], out_acc.at[pl.ds(0, S1_BATCH)], sem_sa.at[s]).wait()
            pltpu.make_async_copy(
                adb.at[s], s_acc.at[pl.ds(0, S1_BATCH)], sem_sa.at[s]).wait()

        @pl.loop(0, S1_CPW)
        def _(t):
            cid = wid * S1_CPW + t
            g0 = cid * IGRP
            pltpu.sync_copy(src_e.at[pl.ds(g0, IGRP)], srcv)
            pltpu.sync_copy(dst_e.at[pl.ds(g0, IGRP)], dstv)

            issue_gathers(0, 0)
            issue_gathers(1, 1)

            @pl.loop(0, S1_BPC)
            def _(b):
                s = lax.rem(b, S1_SETS)
                sn = lax.rem(b + 2, S1_SETS)

                @pl.when(b + 2 < S1_BPC)
                def _():
                    @pl.when(b >= 2)
                    def _():
                        wait_scatters(sn)
                    issue_gathers(b + 2, sn)

                row = lax.shift_right_logical(b, 3)
                off = lax.bitwise_and(b, IGRP - 1) * LANES
                dreg = dstv[row, pl.ds(off, LANES)]
                wait_gathers(s)

                sv = jnp.full((LANES,), s, jnp.int32)

                @plsc.parallel_loop(0, S1_BATCH, unroll=4)
                def _(i):
                    x = (asb[s, i, pl.ds(0, LANES)]
                         + adb[s, i, pl.ds(0, LANES)])
                    w16 = jnp.exp(_leaky(x) - mreg)
                    iv = jnp.full((LANES,), i, jnp.int32)
                    dsts = dreg.at[iv].get(mode="promise_in_bounds")
                    col = (lax.bitwise_and(dsts, 7) * LANES) + lanes_iota
                    for blk in range(H):
                        adb[s, i, pl.ds(blk * LANES, LANES)] = zero
                    plsc.store_scatter(adb, [sv, iv, col], w16)
                    for j in range(H):
                        jv = jnp.full((LANES,), j, jnp.int32)
                        wspl = w16.at[jv].get(mode="promise_in_bounds")
                        sl = pl.ds(j * DH, DH)
                        msgb[s, i, sl] = msgb[s, i, sl] * wspl

                d8 = lax.shift_right_logical(dreg, 3)
                pltpu.async_copy(msgb.at[s], out_acc.at[dreg],
                                 sem_sa.at[s], add=True)
                pltpu.async_copy(adb.at[s], s_acc.at[d8],
                                 sem_sa.at[s], add=True)

            for sst in range(S1_SETS):
                wait_scatters(jnp.int32(sst))

        plsc.subcore_barrier()

        @pl.loop(0, -(-(DBLOCKS + DBLOCKS_S) // NS))
        def _(t):
            b = t * NS + sub

            @pl.when(b < DBLOCKS)
            def _():
                r = b * DROWS
                pltpu.sync_copy(out_acc.at[pl.ds(r, DROWS)],
                                p_out.at[core, pl.ds(r, DROWS)])

            @pl.when(jnp.logical_and(b >= DBLOCKS, b < DBLOCKS + DBLOCKS_S))
            def _():
                r = (b - DBLOCKS) * DROWS
                pltpu.sync_copy(s_acc.at[pl.ds(r, DROWS)],
                                s_out.at[core, pl.ds(r, DROWS)])

        plsc.subcore_barrier()


def _t2a_body(p0_ref, p1_ref, s0_ref, s1_ref, exp_ref,
              o_ref, sum1_ref, sum2_ref):
    i = pl.program_id(0)
    t = p0_ref[0] + p1_ref[0]
    ssum = s0_ref[0] + s1_ref[0]
    den = jnp.dot(ssum, exp_ref[...], preferred_element_type=jnp.float32)
    o = jnp.maximum(t / (den + 1e-16), 0.0)
    o_ref[...] = o
    b1 = jnp.sum(o, axis=0, keepdims=True)
    b2 = jnp.sum(o * o, axis=0, keepdims=True)

    @pl.when(i == 0)
    def _():
        sum1_ref[...] = b1
        sum2_ref[...] = b2

    @pl.when(i != 0)
    def _():
        sum1_ref[...] = sum1_ref[...] + b1
        sum2_ref[...] = sum2_ref[...] + b2


def _t2a_call(p, s, expand):
    return pl.pallas_call(
        _t2a_body,
        grid=(N // TBLK,),
        in_specs=[
            pl.BlockSpec((1, TBLK, D), lambda i: (0, i, 0)),
            pl.BlockSpec((1, TBLK, D), lambda i: (1, i, 0)),
            pl.BlockSpec((1, TBLK, LANES), lambda i: (0, i, 0)),
            pl.BlockSpec((1, TBLK, LANES), lambda i: (1, i, 0)),
            pl.BlockSpec((LANES, D), lambda i: (0, 0)),
        ],
        out_specs=[
            pl.BlockSpec((TBLK, D), lambda i: (i, 0)),
            pl.BlockSpec((1, D), lambda i: (0, 0)),
            pl.BlockSpec((1, D), lambda i: (0, 0)),
        ],
        out_shape=[
            jax.ShapeDtypeStruct((N, D), jnp.float32),
            jax.ShapeDtypeStruct((1, D), jnp.float32),
            jax.ShapeDtypeStruct((1, D), jnp.float32),
        ],
    )(p, p, s, s, expand)


def _t2b_body(o_ref, a_ref, b_ref, out_ref):
    out_ref[...] = o_ref[...] * a_ref[...] + b_ref[...]


def _bn_apply(o, a, b):
    return pl.pallas_call(
        _t2b_body,
        grid=(N // TBLK,),
        in_specs=[
            pl.BlockSpec((TBLK, D), lambda i: (i, 0)),
            pl.BlockSpec((1, D), lambda i: (0, 0)),
            pl.BlockSpec((1, D), lambda i: (0, 0)),
        ],
        out_specs=pl.BlockSpec((TBLK, D), lambda i: (i, 0)),
        out_shape=jax.ShapeDtypeStruct((N, D), jnp.float32),
    )(o, a, b)


@functools.cache
def _s2_kernel():
    return pl.kernel(
        _s2_score,
        out_type=jax.ShapeDtypeStruct((EL_PAD,), jnp.float32),
        mesh=_sc_mesh(),
        scratch_types=[
            pltpu.VMEM((IGRP, GRP), jnp.int32),
            pltpu.VMEM((IGRP, GRP), jnp.int32),
            pltpu.VMEM((S2_QTR, D), jnp.float32),
            pltpu.VMEM((S2_QTR, D), jnp.float32),
            pltpu.VMEM((S2_QTR, LANES), jnp.float32),
            pltpu.VMEM((S2_QTR,), jnp.float32),
            pltpu.SemaphoreType.DMA,
            pltpu.SemaphoreType.DMA,
        ],
        compiler_params=pltpu.CompilerParams(needs_layout_passes=False),
    )


def _s2_score(nr_hbm, ncd_hbm, i0_hbm, i1_hbm, out_hbm,
              i0, i1, r_rows, c_rows, tbuf, outbuf, sem1, sem2):
    core = lax.axis_index("core")
    sub = lax.axis_index("subcore")
    wid = sub * NC + core
    lanes_iota = lax.iota(jnp.int32, LANES)

    @pl.loop(0, S2_ITERS)
    def _(t):
        cid = t * NW + wid

        @pl.when(cid < S2_NCHUNKS)
        def _():
            g0 = cid * IGRP
            pltpu.sync_copy(i0_hbm.at[pl.ds(g0, IGRP)], i0)
            pltpu.sync_copy(i1_hbm.at[pl.ds(g0, IGRP)], i1)
            for q in range(4):
                cps = []
                for g in range(2):
                    gi = q * 2 + g
                    sl = pl.ds(g * GRP, GRP)
                    cps.append(pltpu.async_copy(nr_hbm.at[i0.at[gi]],
                                                r_rows.at[sl], sem1))
                    cps.append(pltpu.async_copy(ncd_hbm.at[i1.at[gi]],
                                                c_rows.at[sl], sem2))
                for cp in cps:
                    cp.wait()

                @pl.loop(0, S2_QTR)
                def _(i):
                    acc = (r_rows[i, pl.ds(0, LANES)]
                           * c_rows[i, pl.ds(0, LANES)])
                    for j in range(1, H):
                        sl = pl.ds(j * DH, LANES)
                        acc = acc + r_rows[i, sl] * c_rows[i, sl]
                    tbuf[i, :] = acc

                @pl.loop(0, S2_QTR, step=LANES)
                def _(g):
                    rows = lanes_iota + g
                    acc = plsc.load_gather(
                        tbuf, [rows, jnp.zeros((LANES,), jnp.int32)])
                    for c in range(1, LANES):
                        acc = acc + plsc.load_gather(
                            tbuf, [rows, jnp.full((LANES,), c, jnp.int32)])
                    outbuf[pl.ds(g, LANES)] = 1.0 / (1.0 + jnp.exp(-acc))

                pltpu.sync_copy(
                    outbuf,
                    out_hbm.at[pl.ds(cid * S2_CHUNK + q * S2_QTR, S2_QTR)])


def _amat(att):
    a3 = att.reshape(H, DH)
    eye = jnp.eye(H, D, dtype=jnp.float32)
    return (a3[:, :, None] * eye[:, None, :]).reshape(H * DH, D)


def kernel(x_req, x_code, edge_index, edge_index_rev, edge_label_index,
           W_req, b_req, W_code, b_code,
           att_src_fwd, att_dst_fwd, att_src_rev, att_dst_rev,
           q_sem, Wk, bk, bn_w, bn_b):
    f32 = jnp.float32

    am_req_1 = _amat(att_src_fwd)
    am_req_2 = _amat(att_dst_rev)
    am_code_1 = _amat(att_dst_fwd)
    am_code_2 = _amat(att_src_rev)

    h_req, asf, adr, max_asf, max_adr = _project(
        x_req, W_req, b_req.reshape(1, D), am_req_1, am_req_2)
    h_code, adf, asr, max_adf, max_asr = _project(
        x_code, W_code, b_code.reshape(1, D), am_code_1, am_code_2)

    m_fwd = _leaky(max_asf + max_adf).reshape(D)
    m_rev = _leaky(max_asr + max_adr).reshape(D)

    pad_n = ((0, NP - N), (0, 0))
    h_req_p = jnp.pad(h_req, pad_n)
    h_code_p = jnp.pad(h_code, pad_n)
    asf_p = jnp.pad(asf, pad_n)
    adf_p = jnp.pad(adf, pad_n)
    asr_p = jnp.pad(asr, pad_n)
    adr_p = jnp.pad(adr, pad_n)

    def edges2d(v):
        v = jnp.pad(v, (0, E_PAD - E), constant_values=N)
        return v.reshape(E_PAD // GRP, GRP)

    srcf = edges2d(edge_index[0])
    dstf = edges2d(edge_index[1])
    srcr = edges2d(edge_index_rev[0])
    dstr = edges2d(edge_index_rev[1])

    pf, sf, pr, sr = _s1_kernel()(h_req_p, h_code_p, asf_p, adf_p, asr_p,
                                  adr_p, srcf, dstf, srcr, dstr,
                                  m_fwd, m_rev)

    def unpack_s(s):
        return s.reshape(NC, NSR * H, LANES)[:, :N, :]

    sf_u = unpack_s(sf)
    sr_u = unpack_s(sr)

    expand = jnp.concatenate(
        [jnp.eye(H, dtype=f32), jnp.zeros((LANES - H, H), f32)], axis=0)
    expand = jnp.repeat(expand, DH, axis=1)

    out_code, c_sum1, c_sum2 = _t2a_call(pf, sf_u, expand)
    out_req, r_sum1, r_sum2 = _t2a_call(pr, sr_u, expand)

    def bn_coeffs(s1, s2):
        mu = s1 / N
        var = s2 / N - mu * mu
        a = bn_w.reshape(1, D) / jnp.sqrt(var + 1e-5)
        b = bn_b.reshape(1, D) - mu * a
        return a, b

    a_r, b_r = bn_coeffs(r_sum1, r_sum2)
    a_c, b_c = bn_coeffs(c_sum1, c_sum2)

    n_req = _bn_apply(out_req, a_r, b_r)
    n_code = _bn_apply(out_code, a_c, b_c)

    eli = jnp.pad(edge_label_index, ((0, 0), (0, EL_PAD - EL)))
    i0 = eli[0].reshape(EL_PAD // GRP, GRP)
    i1 = eli[1].reshape(EL_PAD // GRP, GRP)

    scores = _s2_kernel()(n_req, n_code, i0, i1)
    return scores[:EL]
